# baseline (speedup 1.0000x reference)
import functools

import jax
import jax.numpy as jnp
from jax import lax
from jax.experimental import pallas as pl
from jax.experimental.pallas import tpu as pltpu
from jax.experimental.pallas import tpu_sc as plsc

N_EMBED = 128
BATCH = 4096
HIST = 50
NC = 2
NS = 16
NW = NC * NS
NCH = 4
CB = BATCH // NCH
BPW = CB // NW
K = 8
NSB = BPW // K
NBUF = 2

_mesh = plsc.VectorSubcoreMesh(core_axis_name="c", subcore_axis_name="s")


@functools.partial(
    pl.kernel,
    out_type=jax.ShapeDtypeStruct((CB, HIST, N_EMBED), jnp.float32),
    mesh=_mesh,
    scratch_types=[
        pltpu.VMEM((BPW, HIST), jnp.int32),
        [pltpu.VMEM((K, HIST, N_EMBED), jnp.float32) for _ in range(NBUF)],
        [pltpu.SemaphoreType.DMA for _ in range(NBUF)],
        [pltpu.SemaphoreType.DMA for _ in range(NBUF)],
    ],
)
def _gather_kernel(idx_hbm, table_hbm, out_hbm, idx_v, bufs, gsems, psems):
    wid = lax.axis_index("s") * NC + lax.axis_index("c")
    b0 = wid * BPW
    pltpu.sync_copy(idx_hbm.at[pl.ds(b0, BPW)], idx_v)

    def fire(s, r):
        for i in range(K):
            pltpu.async_copy(table_hbm.at[idx_v.at[s * K + i]], bufs[r].at[i],
                             gsems[r])

    def drain(s, r):
        for i in range(K):
            pltpu.make_async_copy(table_hbm.at[idx_v.at[s * K + i]],
                                  bufs[r].at[i], gsems[r]).wait()

    for r in range(NBUF):
        fire(r, r)

    def outer(t, carry):
        for r in range(NBUF):
            s = t * NBUF + r
            drain(s, r)
            pltpu.async_copy(bufs[r], out_hbm.at[pl.ds(b0 + s * K, K)],
                             psems[r]).wait()

            @pl.when(s + NBUF < NSB)
            def _():
                fire(s + NBUF, r)

        return carry

    lax.fori_loop(0, NSB // NBUF, outer, 0)


def kernel(input, embedding):
    idx = input.astype(jnp.int32)
    outs = [_gather_kernel(idx[c * CB:(c + 1) * CB], embedding)
            for c in range(NCH)]
    return jnp.concatenate(outs, axis=0)


# speedup vs baseline: 1.8032x; 1.8032x over previous
import functools

import jax
import jax.numpy as jnp
from jax import lax
from jax.experimental import pallas as pl
from jax.experimental.pallas import tpu as pltpu
from jax.experimental.pallas import tpu_sc as plsc

N_EMBED = 128
BATCH = 4096
HIST = 50
NC = 2
NS = 16
NW = NC * NS
NCH = 1
CB = BATCH // NCH
BPW = CB // NW
K = 8
NSB = BPW // K
NBUF = 2

_mesh = plsc.VectorSubcoreMesh(core_axis_name="c", subcore_axis_name="s")


@functools.partial(
    pl.kernel,
    out_type=jax.ShapeDtypeStruct((CB, HIST, N_EMBED), jnp.float32),
    mesh=_mesh,
    compiler_params=pltpu.CompilerParams(use_tc_tiling_on_sc=True),
    scratch_types=[
        pltpu.VMEM((BPW, HIST), jnp.int32),
        [pltpu.VMEM((K, HIST, N_EMBED), jnp.float32) for _ in range(NBUF)],
        [pltpu.SemaphoreType.DMA for _ in range(NBUF)],
        [pltpu.SemaphoreType.DMA for _ in range(NBUF)],
    ],
)
def _gather_kernel(idx_hbm, table_hbm, out_hbm, idx_v, bufs, gsems, psems):
    wid = lax.axis_index("s") * NC + lax.axis_index("c")
    b0 = wid * BPW
    pltpu.sync_copy(idx_hbm.at[pl.ds(b0, BPW)], idx_v)

    def fire(s, r):
        for i in range(K):
            pltpu.async_copy(table_hbm.at[idx_v.at[s * K + i]], bufs[r].at[i],
                             gsems[r])

    def drain(s, r):
        for i in range(K):
            pltpu.make_async_copy(table_hbm.at[idx_v.at[s * K + i]],
                                  bufs[r].at[i], gsems[r]).wait()

    for r in range(NBUF):
        fire(r, r)

    def outer(t, carry):
        for r in range(NBUF):
            s = t * NBUF + r
            drain(s, r)
            pltpu.async_copy(bufs[r], out_hbm.at[pl.ds(b0 + s * K, K)],
                             psems[r]).wait()

            @pl.when(s + NBUF < NSB)
            def _():
                fire(s + NBUF, r)

        return carry

    lax.fori_loop(0, NSB // NBUF, outer, 0)


def kernel(input, embedding):
    return _gather_kernel(input.astype(jnp.int32), embedding)
